# Initial kernel scaffold; baseline (speedup 1.0000x reference)
import jax
import jax.numpy as jnp
from jax.experimental import pallas as pl


def kernel(tokens_id, W, b, gate, binary_table):
    raise NotImplementedError("write your pallas kernel here")


# speedup vs baseline: 185.9664x; 185.9664x over previous
import jax
import jax.numpy as jnp
from jax.experimental import pallas as pl

NT = 8


def _bias_kernel(ids_ref, w_ref, b_ref, gate_ref, table_ref, out_ref):
    ids = jnp.clip(ids_ref[0, 0, :], 0, NT - 1)
    T = ids.shape[0]
    iota = jax.lax.broadcasted_iota(jnp.int32, (T, NT), 1)
    onehot = (ids[:, None] == iota).astype(jnp.float32)
    rows = jax.lax.dot(onehot, table_ref[:, :],
                       preferred_element_type=jnp.float32)
    vals = jax.lax.dot(rows, onehot.T, preferred_element_type=jnp.float32)
    tg = jnp.tanh(gate_ref[0, 0])
    scale = tg * w_ref[0, :]
    offset = tg * b_ref[0, :]
    out_ref[0, :, :, :] = (vals[None, :, :] * scale[:, None, None]
                           + offset[:, None, None])


def kernel(tokens_id, W, b, gate, binary_table):
    B, T = tokens_id.shape
    H = W.shape[0]
    ids3 = tokens_id.reshape(B, 1, T)
    w2 = W.reshape(1, H)
    b2 = b.reshape(1, H)
    gate2 = gate.reshape(1, 1)
    return pl.pallas_call(
        _bias_kernel,
        grid=(B,),
        in_specs=[
            pl.BlockSpec((1, 1, T), lambda i: (i, 0, 0)),
            pl.BlockSpec((1, H), lambda i: (0, 0)),
            pl.BlockSpec((1, H), lambda i: (0, 0)),
            pl.BlockSpec((1, 1), lambda i: (0, 0)),
            pl.BlockSpec((NT, NT), lambda i: (0, 0)),
        ],
        out_specs=pl.BlockSpec((1, H, T, T), lambda i: (i, 0, 0, 0)),
        out_shape=jax.ShapeDtypeStruct((B, H, T, T), jnp.float32),
    )(ids3, w2, b2, gate2, binary_table)


# speedup vs baseline: 381.3747x; 2.0508x over previous
import jax
import jax.numpy as jnp
from jax.experimental import pallas as pl

NT = 8


BB = 4


def _bias_kernel(ids_ref, w_ref, b_ref, gate_ref, table_ref, out_ref):
    ids = jnp.clip(ids_ref[:, 0, :], 0, NT - 1)
    bb, T = ids.shape
    iota = jax.lax.broadcasted_iota(jnp.int32, (bb, T, NT), 2)
    onehot = (ids[:, :, None] == iota).astype(jnp.float32)
    rows = jax.lax.dot_general(
        onehot, table_ref[:, :],
        dimension_numbers=(((2,), (0,)), ((), ())),
        preferred_element_type=jnp.float32)
    vals = jax.lax.dot_general(
        rows, onehot,
        dimension_numbers=(((2,), (2,)), ((0,), (0,))),
        preferred_element_type=jnp.float32)
    tg = jnp.tanh(gate_ref[0, 0])
    scale = tg * w_ref[0, :]
    offset = tg * b_ref[0, :]
    out_ref[:, :, :, :] = (vals[:, None, :, :] * scale[None, :, None, None]
                           + offset[None, :, None, None])


def kernel(tokens_id, W, b, gate, binary_table):
    B, T = tokens_id.shape
    H = W.shape[0]
    ids3 = tokens_id.reshape(B, 1, T)
    w2 = W.reshape(1, H)
    b2 = b.reshape(1, H)
    gate2 = gate.reshape(1, 1)
    return pl.pallas_call(
        _bias_kernel,
        grid=(B // BB,),
        in_specs=[
            pl.BlockSpec((BB, 1, T), lambda i: (i, 0, 0)),
            pl.BlockSpec((1, H), lambda i: (0, 0)),
            pl.BlockSpec((1, H), lambda i: (0, 0)),
            pl.BlockSpec((1, 1), lambda i: (0, 0)),
            pl.BlockSpec((NT, NT), lambda i: (0, 0)),
        ],
        out_specs=pl.BlockSpec((BB, H, T, T), lambda i: (i, 0, 0, 0)),
        out_shape=jax.ShapeDtypeStruct((B, H, T, T), jnp.float32),
    )(ids3, w2, b2, gate2, binary_table)


# speedup vs baseline: 414.8609x; 1.0878x over previous
import jax
import jax.numpy as jnp
from jax.experimental import pallas as pl

NT = 8


BB = 8


def _bias_kernel(ids_ref, w_ref, b_ref, gate_ref, table_ref, out_ref):
    ids = jnp.clip(ids_ref[:, 0, :], 0, NT - 1)
    bb, T = ids.shape
    iota = jax.lax.broadcasted_iota(jnp.int32, (bb, T, NT), 2)
    onehot = (ids[:, :, None] == iota).astype(jnp.float32)
    rows = jax.lax.dot_general(
        onehot, table_ref[:, :],
        dimension_numbers=(((2,), (0,)), ((), ())),
        preferred_element_type=jnp.float32)
    vals = jax.lax.dot_general(
        rows, onehot,
        dimension_numbers=(((2,), (2,)), ((0,), (0,))),
        preferred_element_type=jnp.float32)
    tg = jnp.tanh(gate_ref[0, 0])
    scale = tg * w_ref[0, :]
    offset = tg * b_ref[0, :]
    out_ref[:, :, :, :] = (vals[:, None, :, :] * scale[None, :, None, None]
                           + offset[None, :, None, None])


def kernel(tokens_id, W, b, gate, binary_table):
    B, T = tokens_id.shape
    H = W.shape[0]
    ids3 = tokens_id.reshape(B, 1, T)
    w2 = W.reshape(1, H)
    b2 = b.reshape(1, H)
    gate2 = gate.reshape(1, 1)
    return pl.pallas_call(
        _bias_kernel,
        grid=(B // BB,),
        in_specs=[
            pl.BlockSpec((BB, 1, T), lambda i: (i, 0, 0)),
            pl.BlockSpec((1, H), lambda i: (0, 0)),
            pl.BlockSpec((1, H), lambda i: (0, 0)),
            pl.BlockSpec((1, 1), lambda i: (0, 0)),
            pl.BlockSpec((NT, NT), lambda i: (0, 0)),
        ],
        out_specs=pl.BlockSpec((BB, H, T, T), lambda i: (i, 0, 0, 0)),
        out_shape=jax.ShapeDtypeStruct((B, H, T, T), jnp.float32),
    )(ids3, w2, b2, gate2, binary_table)
